# baseline (speedup 1.0000x reference)
import functools

import jax
import jax.numpy as jnp
from jax import lax
from jax.experimental import pallas as pl
from jax.experimental.pallas import tpu as pltpu
from jax.experimental.pallas import tpu_sc as plsc

NUM_LABELS = 32
PAD = 0
NEG = -1000000000.0
QT = 256


def _sc_gather(E2, idx):
    info = plsc.get_sparse_core_info()
    NC, NS = info.num_cores, info.num_subcores
    NW = NC * NS
    B = idx.shape[0]
    D = E2.shape[1]
    b_w = B // NW
    CH = 4
    CB = b_w // CH

    mesh = plsc.VectorSubcoreMesh(core_axis_name="c", subcore_axis_name="s")
    scratch = ([pltpu.VMEM((CB,), jnp.int32) for _ in range(CH)]
               + [pltpu.VMEM((CB, D), jnp.float32) for _ in range(CH)]
               + [pltpu.SemaphoreType.DMA])

    @functools.partial(
        pl.kernel,
        mesh=mesh,
        out_type=jax.ShapeDtypeStruct((B, D), jnp.float32),
        scratch_types=scratch,
    )
    def gather_kernel(table_hbm, idx_hbm, out_hbm, *refs):
        idx_refs = refs[:CH]
        row_refs = refs[CH:2 * CH]
        sem = refs[2 * CH]
        wid = lax.axis_index("s") * NC + lax.axis_index("c")
        base = wid * b_w
        for c in range(CH):
            pltpu.sync_copy(idx_hbm.at[pl.ds(base + c * CB, CB)], idx_refs[c])
        handles = [
            pltpu.async_copy(table_hbm.at[idx_refs[c]], row_refs[c], sem)
            for c in range(CH)
        ]
        for h in handles:
            h.wait()
        for c in range(CH):
            pltpu.sync_copy(row_refs[c], out_hbm.at[pl.ds(base + c * CB, CB)])

    return gather_kernel(E2, idx)


def _halves(pair, par):
    H = pair.shape[1] // 2
    return jnp.where(par != 0.0, pair[:, H:], pair[:, :H])


def _decode_body(x_ref, sup_ref, spar_ref, lab_ref, qtok_ref, qpar_ref,
                 best_ref, near_ref, yn_s, y2_s):
    @pl.when(pl.program_id(0) == 0)
    def _():
        s = _halves(sup_ref[...], spar_ref[...])
        ns = jnp.sqrt(jnp.sum(s * s, axis=1, keepdims=True))
        yn = s / jnp.maximum(ns, 1e-12)
        yn_s[...] = yn
        y2 = jnp.sum(yn * yn, axis=1, keepdims=True)
        y2_s[...] = jnp.transpose(y2)

    x = _halves(x_ref[...], qpar_ref[...])
    nx = jnp.sqrt(jnp.sum(x * x, axis=1, keepdims=True))
    xn = x / jnp.maximum(nx, 1e-12)
    x2 = jnp.sum(xn * xn, axis=1, keepdims=True)

    d = lax.dot_general(xn, yn_s[...], (((1,), (1,)), ((), ())),
                        preferred_element_type=jnp.float32)
    scores = 2.0 * d - x2 - y2_s[...]

    lab = lab_ref[...]
    qv = qtok_ref[...] != float(PAD)
    lv = lab != float(PAD)
    scores = jnp.where(jnp.logical_and(qv, lv), scores, NEG)

    m = jnp.max(scores, axis=1, keepdims=True)
    iota = lax.broadcasted_iota(jnp.int32, scores.shape, 1)
    best = jnp.min(jnp.where(scores == m, iota, jnp.int32(2**30)),
                   axis=1, keepdims=True)
    bl = jnp.max(jnp.where(iota == best, lab, 0.0), axis=1, keepdims=True)
    best_ref[...] = bl.astype(jnp.int32)

    sbf = scores.astype(jnp.bfloat16)
    negb = jnp.bfloat16(NEG)
    cols = []
    for l in range(NUM_LABELS):
        sel = jnp.where(lab == float(l), sbf, negb)
        cols.append(jnp.max(sel, axis=1, keepdims=True))
    near = jnp.concatenate(cols, axis=1).astype(jnp.float32)
    near_ref[...] = jnp.where(near < NEG * 0.5, NEG, near)


def _decode(emb, spar, labels_f, qtok_f, qpar, S, interpret=False):
    Q = qtok_f.shape[0]
    W = emb.shape[1]
    grid = (Q // QT,)
    sup_block = Q // S
    return pl.pallas_call(
        _decode_body,
        grid=grid,
        in_specs=[
            pl.BlockSpec((QT, W), lambda i: (i, 0)),
            pl.BlockSpec((S, W), lambda i, b=sup_block: (b, 0)),
            pl.BlockSpec((S, 1), lambda i: (0, 0)),
            pl.BlockSpec((1, S), lambda i: (0, 0)),
            pl.BlockSpec((QT, 1), lambda i: (i, 0)),
            pl.BlockSpec((QT, 1), lambda i: (i, 0)),
        ],
        out_specs=[
            pl.BlockSpec((QT, 1), lambda i: (i, 0)),
            pl.BlockSpec((QT, NUM_LABELS), lambda i: (i, 0)),
        ],
        out_shape=[
            jax.ShapeDtypeStruct((Q, 1), jnp.int32),
            jax.ShapeDtypeStruct((Q, NUM_LABELS), jnp.float32),
        ],
        scratch_shapes=[
            pltpu.VMEM((S, W // 2), jnp.float32),
            pltpu.VMEM((1, S), jnp.float32),
        ],
        interpret=interpret,
    )(emb, emb, spar, labels_f, qtok_f, qpar)


def kernel(support, label_support, query, E):
    support = support.astype(jnp.int32)
    query_i = query.astype(jnp.int32)
    qflat = query_i.reshape(-1)
    S = support.shape[0]
    H = E.shape[1]

    E2 = E.reshape(E.shape[0] // 2, 2 * H)
    idx = jnp.concatenate([qflat, support])
    par_f = (idx & 1).astype(jnp.float32).reshape(-1, 1)
    emb = _sc_gather(E2, idx >> 1)

    Q = qflat.shape[0]
    labels_f = label_support.astype(jnp.float32).reshape(1, S)
    qtok_f = qflat.astype(jnp.float32).reshape(-1, 1)

    best, near = _decode(emb, par_f[Q:], labels_f, qtok_f, par_f[:Q], S)
    return (best.reshape(query.shape),
            near.reshape(query.shape + (NUM_LABELS,)))


# speedup vs baseline: 1.0078x; 1.0078x over previous
import functools

import jax
import jax.numpy as jnp
from jax import lax
from jax.experimental import pallas as pl
from jax.experimental.pallas import tpu as pltpu
from jax.experimental.pallas import tpu_sc as plsc

NUM_LABELS = 32
PAD = 0
NEG = -1000000000.0
QT = 256
LANES = 16


def _sc_gather(E2, qtok, stok):
    info = plsc.get_sparse_core_info()
    NC, NS = info.num_cores, info.num_subcores
    NW = NC * NS
    Q = qtok.shape[0]
    S = stok.shape[0]
    D = E2.shape[1]
    q_w = Q // NW
    s_w = S // NW
    QCH = q_w // 128

    mesh = plsc.VectorSubcoreMesh(core_axis_name="c", subcore_axis_name="s")
    chunk_sizes = [128] * QCH + [s_w]
    scratch = ([pltpu.VMEM((c,), jnp.int32) for c in chunk_sizes]
               + [pltpu.VMEM((c, D), jnp.float32) for c in chunk_sizes]
               + [pltpu.SemaphoreType.DMA])
    NCHUNK = len(chunk_sizes)

    @functools.partial(
        pl.kernel,
        mesh=mesh,
        out_type=jax.ShapeDtypeStruct((Q + S, D), jnp.float32),
        scratch_types=scratch,
    )
    def gather_kernel(table_hbm, qtok_hbm, stok_hbm, out_hbm, *refs):
        idx_refs = refs[:NCHUNK]
        row_refs = refs[NCHUNK:2 * NCHUNK]
        sem = refs[2 * NCHUNK]
        wid = lax.axis_index("s") * NC + lax.axis_index("c")
        qbase = wid * q_w
        sbase = wid * s_w
        srcs = ([(qtok_hbm, qbase + c * 128, qbase + c * 128)
                 for c in range(QCH)]
                + [(stok_hbm, sbase, Q + sbase)])
        for (src, off, _), cs, iref in zip(srcs, chunk_sizes, idx_refs):
            pltpu.sync_copy(src.at[pl.ds(off, cs)], iref)
            for v in range(cs // LANES):
                sl = pl.ds(v * LANES, LANES)
                iref[sl] = iref[sl] >> 1
        handles = [
            pltpu.async_copy(table_hbm.at[iref], rref, sem)
            for iref, rref in zip(idx_refs, row_refs)
        ]
        for h in handles:
            h.wait()
        for (_, _, out_off), cs, rref in zip(srcs, chunk_sizes, row_refs):
            pltpu.sync_copy(rref, out_hbm.at[pl.ds(out_off, cs)])

    return gather_kernel(E2, qtok, stok)


def _halves(pair, par):
    H = pair.shape[1] // 2
    return jnp.where(par, pair[:, H:], pair[:, :H])


def _decode_body(x_ref, sup_ref, stok_ref, lab_ref, qtok_ref,
                 best_ref, near_ref, ynT_s, y2_s):
    @pl.when(pl.program_id(0) == 0)
    def _():
        s = _halves(sup_ref[...], (stok_ref[...] & 1) != 0)
        ns = jnp.sqrt(jnp.sum(s * s, axis=1, keepdims=True))
        yn = s / jnp.maximum(ns, 1e-12)
        ynT = jnp.transpose(yn)
        ynT_s[...] = ynT
        y2_s[...] = jnp.sum(ynT * ynT, axis=0, keepdims=True)

    qtok = qtok_ref[...]
    x = _halves(x_ref[...], (qtok & 1) != 0)
    nx = jnp.sqrt(jnp.sum(x * x, axis=1, keepdims=True))
    xn = x / jnp.maximum(nx, 1e-12)
    x2 = jnp.sum(xn * xn, axis=1, keepdims=True)

    d = lax.dot_general(xn, ynT_s[...], (((1,), (0,)), ((), ())),
                        preferred_element_type=jnp.float32)
    scores = 2.0 * d - x2 - y2_s[...]

    lab = lab_ref[...]
    qv = qtok != PAD
    lv = lab != float(PAD)
    scores = jnp.where(jnp.logical_and(qv, lv), scores, NEG)

    m = jnp.max(scores, axis=1, keepdims=True)
    iota = lax.broadcasted_iota(jnp.int32, scores.shape, 1)
    best = jnp.min(jnp.where(scores == m, iota, jnp.int32(2**30)),
                   axis=1, keepdims=True)
    bl = jnp.max(jnp.where(iota == best, lab, 0.0), axis=1, keepdims=True)
    best_ref[...] = bl.astype(jnp.int32)

    sbf = scores.astype(jnp.bfloat16)
    negb = jnp.bfloat16(NEG)
    cols = []
    for l in range(NUM_LABELS):
        sel = jnp.where(lab == float(l), sbf, negb)
        cols.append(jnp.max(sel, axis=1, keepdims=True))
    near = jnp.concatenate(cols, axis=1).astype(jnp.float32)
    near_ref[...] = jnp.where(near < NEG * 0.5, NEG, near)


def _decode(emb, stok_c, labels_f, qtok_c, S, interpret=False):
    Q = qtok_c.shape[0]
    W = emb.shape[1]
    grid = (Q // QT,)
    sup_block = Q // S
    return pl.pallas_call(
        _decode_body,
        grid=grid,
        in_specs=[
            pl.BlockSpec((QT, W), lambda i: (i, 0)),
            pl.BlockSpec((S, W), lambda i, b=sup_block: (b, 0)),
            pl.BlockSpec((S, 1), lambda i: (0, 0)),
            pl.BlockSpec((1, S), lambda i: (0, 0)),
            pl.BlockSpec((QT, 1), lambda i: (i, 0)),
        ],
        out_specs=[
            pl.BlockSpec((QT, 1), lambda i: (i, 0)),
            pl.BlockSpec((QT, NUM_LABELS), lambda i: (i, 0)),
        ],
        out_shape=[
            jax.ShapeDtypeStruct((Q, 1), jnp.int32),
            jax.ShapeDtypeStruct((Q, NUM_LABELS), jnp.float32),
        ],
        scratch_shapes=[
            pltpu.VMEM((W // 2, S), jnp.float32),
            pltpu.VMEM((1, S), jnp.float32),
        ],
        interpret=interpret,
    )(emb, emb, stok_c, labels_f, qtok_c)


def kernel(support, label_support, query, E):
    support = support.astype(jnp.int32)
    qflat = query.astype(jnp.int32).reshape(-1)
    S = support.shape[0]
    H = E.shape[1]

    E2 = E.reshape(E.shape[0] // 2, 2 * H)
    emb = _sc_gather(E2, qflat, support)

    labels_f = label_support.astype(jnp.float32).reshape(1, S)
    best, near = _decode(emb, support.reshape(S, 1), labels_f,
                         qflat.reshape(-1, 1), S)
    return (best.reshape(query.shape),
            near.reshape(query.shape + (NUM_LABELS,)))


# speedup vs baseline: 1.0854x; 1.0769x over previous
import functools

import jax
import jax.numpy as jnp
from jax import lax
from jax.experimental import pallas as pl
from jax.experimental.pallas import tpu as pltpu
from jax.experimental.pallas import tpu_sc as plsc

NUM_LABELS = 32
PAD = 0
NEG = -1000000000.0
QT = 256


def _sc_gather(Epad, qtok, stok):
    info = plsc.get_sparse_core_info()
    NC, NS = info.num_cores, info.num_subcores
    NW = NC * NS
    Q = qtok.shape[0]
    S = stok.shape[0]
    D = Epad.shape[1]
    q_w = Q // NW
    s_w = S // NW
    QCH = q_w // 128

    mesh = plsc.VectorSubcoreMesh(core_axis_name="c", subcore_axis_name="s")
    chunk_sizes = [128] * QCH + [s_w]
    scratch = ([pltpu.VMEM((c,), jnp.int32) for c in chunk_sizes]
               + [pltpu.VMEM((c, D), jnp.float32) for c in chunk_sizes]
               + [pltpu.SemaphoreType.DMA])
    NCHUNK = len(chunk_sizes)

    @functools.partial(
        pl.kernel,
        mesh=mesh,
        out_type=jax.ShapeDtypeStruct((Q + S, D), jnp.float32),
        scratch_types=scratch,
    )
    def gather_kernel(table_hbm, qtok_hbm, stok_hbm, out_hbm, *refs):
        idx_refs = refs[:NCHUNK]
        row_refs = refs[NCHUNK:2 * NCHUNK]
        sem = refs[2 * NCHUNK]
        wid = lax.axis_index("s") * NC + lax.axis_index("c")
        qbase = wid * q_w
        sbase = wid * s_w
        srcs = ([(qtok_hbm, qbase + c * 128, qbase + c * 128)
                 for c in range(QCH)]
                + [(stok_hbm, sbase, Q + sbase)])
        for (src, off, _), cs, iref in zip(srcs, chunk_sizes, idx_refs):
            pltpu.sync_copy(src.at[pl.ds(off, cs)], iref)
        handles = [
            pltpu.async_copy(table_hbm.at[iref], rref, sem)
            for iref, rref in zip(idx_refs, row_refs)
        ]
        for h in handles:
            h.wait()
        for (_, _, out_off), cs, rref in zip(srcs, chunk_sizes, row_refs):
            pltpu.sync_copy(rref, out_hbm.at[pl.ds(out_off, cs)])

    return gather_kernel(Epad, qtok, stok)


def _decode_body(x_ref, sT_ref, lab_ref, qtok_ref, best_ref, near_ref,
                 ynT_s, y2_s):
    @pl.when(pl.program_id(0) == 0)
    def _():
        sT = sT_ref[...]
        ns = jnp.sqrt(jnp.sum(sT * sT, axis=0, keepdims=True))
        ynT = sT / jnp.maximum(ns, 1e-12)
        ynT_s[...] = ynT
        y2_s[...] = jnp.sum(ynT * ynT, axis=0, keepdims=True)

    H = sT_ref.shape[0]
    x = x_ref[:, :H]
    nx = jnp.sqrt(jnp.sum(x * x, axis=1, keepdims=True))
    xn = x / jnp.maximum(nx, 1e-12)
    x2 = jnp.sum(xn * xn, axis=1, keepdims=True)

    d = lax.dot_general(xn, ynT_s[...], (((1,), (0,)), ((), ())),
                        preferred_element_type=jnp.float32)
    scores = 2.0 * d - x2 - y2_s[...]

    lab = lab_ref[...]
    qv = qtok_ref[...] != float(PAD)
    lv = lab != float(PAD)
    scores = jnp.where(jnp.logical_and(qv, lv), scores, NEG)

    m = jnp.max(scores, axis=1, keepdims=True)
    iota = lax.broadcasted_iota(jnp.int32, scores.shape, 1)
    best = jnp.min(jnp.where(scores == m, iota, jnp.int32(2**30)),
                   axis=1, keepdims=True)
    bl = jnp.max(jnp.where(iota == best, lab, 0.0), axis=1, keepdims=True)
    best_ref[...] = bl.astype(jnp.int32)

    sbf = scores.astype(jnp.bfloat16)
    negb = jnp.bfloat16(NEG)
    cols = []
    for l in range(NUM_LABELS):
        sel = jnp.where(lab == float(l), sbf, negb)
        cols.append(jnp.max(sel, axis=1, keepdims=True))
    near = jnp.concatenate(cols, axis=1).astype(jnp.float32)
    near_ref[...] = jnp.where(near < NEG * 0.5, NEG, near)


def _decode(emb, sT, labels_f, qtok_f, interpret=False):
    Q = qtok_f.shape[0]
    W = emb.shape[1]
    H = sT.shape[0]
    S = sT.shape[1]
    grid = (Q // QT,)
    return pl.pallas_call(
        _decode_body,
        grid=grid,
        in_specs=[
            pl.BlockSpec((QT, W), lambda i: (i, 0)),
            pl.BlockSpec((H, S), lambda i: (0, 0)),
            pl.BlockSpec((1, S), lambda i: (0, 0)),
            pl.BlockSpec((QT, 1), lambda i: (i, 0)),
        ],
        out_specs=[
            pl.BlockSpec((QT, 1), lambda i: (i, 0)),
            pl.BlockSpec((QT, NUM_LABELS), lambda i: (i, 0)),
        ],
        out_shape=[
            jax.ShapeDtypeStruct((Q, 1), jnp.int32),
            jax.ShapeDtypeStruct((Q, NUM_LABELS), jnp.float32),
        ],
        scratch_shapes=[
            pltpu.VMEM((H, S), jnp.float32),
            pltpu.VMEM((1, S), jnp.float32),
        ],
        interpret=interpret,
    )(emb, sT, labels_f, qtok_f)


def kernel(support, label_support, query, E):
    support = support.astype(jnp.int32)
    qflat = query.astype(jnp.int32).reshape(-1)
    S = support.shape[0]
    H = E.shape[1]
    Q = qflat.shape[0]

    E_pad = jnp.pad(E, ((0, 0), (0, 128 - H)))
    emb = _sc_gather(E_pad, qflat, support)

    sT = emb[Q:, :H].T
    labels_f = label_support.astype(jnp.float32).reshape(1, S)
    qtok_f = qflat.astype(jnp.float32).reshape(-1, 1)

    best, near = _decode(emb, sT, labels_f, qtok_f)
    return (best.reshape(query.shape),
            near.reshape(query.shape + (NUM_LABELS,)))


# speedup vs baseline: 1.1509x; 1.0604x over previous
import functools

import jax
import jax.numpy as jnp
from jax import lax
from jax.experimental import pallas as pl
from jax.experimental.pallas import tpu as pltpu
from jax.experimental.pallas import tpu_sc as plsc

NUM_LABELS = 32
PAD = 0
NEG = -1000000000.0
QT = 256


def _sc_gather(Epad, qtok, stok):
    info = plsc.get_sparse_core_info()
    NC, NS = info.num_cores, info.num_subcores
    NW = NC * NS
    Q = qtok.shape[0]
    S = stok.shape[0]
    D = Epad.shape[1]
    q_w = Q // NW
    s_w = S // NW
    QCH = q_w // 128

    mesh = plsc.VectorSubcoreMesh(core_axis_name="c", subcore_axis_name="s")
    chunk_sizes = [128] * QCH + [s_w]
    scratch = ([pltpu.VMEM((c,), jnp.int32) for c in chunk_sizes]
               + [pltpu.VMEM((c, D), jnp.float32) for c in chunk_sizes]
               + [pltpu.SemaphoreType.DMA])
    NCHUNK = len(chunk_sizes)

    @functools.partial(
        pl.kernel,
        mesh=mesh,
        out_type=jax.ShapeDtypeStruct((Q + S, D), jnp.float32),
        scratch_types=scratch,
    )
    def gather_kernel(table_hbm, qtok_hbm, stok_hbm, out_hbm, *refs):
        idx_refs = refs[:NCHUNK]
        row_refs = refs[NCHUNK:2 * NCHUNK]
        sem = refs[2 * NCHUNK]
        wid = lax.axis_index("s") * NC + lax.axis_index("c")
        qbase = wid * q_w
        sbase = wid * s_w
        srcs = ([(qtok_hbm, qbase + c * 128, qbase + c * 128)
                 for c in range(QCH)]
                + [(stok_hbm, sbase, Q + sbase)])
        for (src, off, _), cs, iref in zip(srcs, chunk_sizes, idx_refs):
            pltpu.sync_copy(src.at[pl.ds(off, cs)], iref)
        handles = [
            pltpu.async_copy(table_hbm.at[iref], rref, sem)
            for iref, rref in zip(idx_refs, row_refs)
        ]
        for h in handles:
            h.wait()
        for (_, _, out_off), cs, rref in zip(srcs, chunk_sizes, row_refs):
            pltpu.sync_copy(rref, out_hbm.at[pl.ds(out_off, cs)])

    return gather_kernel(Epad, qtok, stok)


def _sc_gather_rows(E, qtok, stok):
    info = plsc.get_sparse_core_info()
    NC, NS = info.num_cores, info.num_subcores
    NW = NC * NS
    Q = qtok.shape[0]
    S = stok.shape[0]
    H = E.shape[1]
    q_w = Q // NW
    s_w = S // NW
    n_w = q_w + s_w
    BATCH = 16

    mesh = plsc.VectorSubcoreMesh(core_axis_name="c", subcore_axis_name="s")
    scratch = [
        pltpu.VMEM((n_w,), jnp.int32),
        pltpu.VMEM((n_w, H), jnp.float32),
        pltpu.SemaphoreType.DMA,
        pltpu.SemaphoreType.DMA,
    ]

    @functools.partial(
        pl.kernel,
        mesh=mesh,
        out_type=jax.ShapeDtypeStruct((Q + S, H), jnp.float32),
        scratch_types=scratch,
    )
    def gather_kernel(table_hbm, qtok_hbm, stok_hbm, out_hbm, idx_v, rows_v,
                      sem0, sem1):
        wid = lax.axis_index("s") * NC + lax.axis_index("c")
        qbase = wid * q_w
        sbase = wid * s_w
        pltpu.sync_copy(qtok_hbm.at[pl.ds(qbase, q_w)],
                        idx_v.at[pl.ds(0, q_w)])
        pltpu.sync_copy(stok_hbm.at[pl.ds(sbase, s_w)],
                        idx_v.at[pl.ds(q_w, s_w)])
        sems = (sem0, sem1)
        pending = []
        for b in range(n_w // BATCH):
            sem = sems[b % 2]
            toks = idx_v[pl.ds(b * BATCH, BATCH)]
            batch = []
            for j in range(BATCH):
                i = b * BATCH + j
                batch.append(pltpu.async_copy(
                    table_hbm.at[pl.ds(toks[j], 1)],
                    rows_v.at[pl.ds(i, 1)], sem))
            for h in pending:
                h.wait()
            pending = batch
        for h in pending:
            h.wait()
        pltpu.sync_copy(rows_v.at[pl.ds(0, q_w)],
                        out_hbm.at[pl.ds(qbase, q_w)])
        pltpu.sync_copy(rows_v.at[pl.ds(q_w, s_w)],
                        out_hbm.at[pl.ds(Q + sbase, s_w)])

    return gather_kernel(E, qtok, stok)


def _decode_body(x_ref, sT_ref, lab_ref, qtok_ref, best_ref, near_ref,
                 ynT_s, y2_s):
    @pl.when(pl.program_id(0) == 0)
    def _():
        sT = sT_ref[...]
        ns = jnp.sqrt(jnp.sum(sT * sT, axis=0, keepdims=True))
        ynT = sT / jnp.maximum(ns, 1e-12)
        ynT_s[...] = ynT
        y2_s[...] = jnp.sum(ynT * ynT, axis=0, keepdims=True)

    H = sT_ref.shape[0]
    x = x_ref[:, :H]
    nx = jnp.sqrt(jnp.sum(x * x, axis=1, keepdims=True))
    xn = x / jnp.maximum(nx, 1e-12)
    x2 = jnp.sum(xn * xn, axis=1, keepdims=True)

    d = lax.dot_general(xn, ynT_s[...], (((1,), (0,)), ((), ())),
                        preferred_element_type=jnp.float32)
    scores = 2.0 * d - x2 - y2_s[...]

    lab = lab_ref[...]
    qv = qtok_ref[...] != float(PAD)
    lv = lab != float(PAD)
    scores = jnp.where(jnp.logical_and(qv, lv), scores, NEG)

    m = jnp.max(scores, axis=1, keepdims=True)
    iota = lax.broadcasted_iota(jnp.int32, scores.shape, 1)
    best = jnp.min(jnp.where(scores == m, iota, jnp.int32(2**30)),
                   axis=1, keepdims=True)
    bl = jnp.max(jnp.where(iota == best, lab, 0.0), axis=1, keepdims=True)
    best_ref[...] = bl.astype(jnp.int32)

    sbf = scores.astype(jnp.bfloat16)
    negb = jnp.bfloat16(NEG)
    cols = []
    for l in range(NUM_LABELS):
        sel = jnp.where(lab == float(l), sbf, negb)
        cols.append(jnp.max(sel, axis=1, keepdims=True))
    near = jnp.concatenate(cols, axis=1).astype(jnp.float32)
    near_ref[...] = jnp.where(near < NEG * 0.5, NEG, near)


def _decode(emb, sT, labels_f, qtok_f, interpret=False):
    Q = qtok_f.shape[0]
    W = emb.shape[1]
    H = sT.shape[0]
    S = sT.shape[1]
    grid = (Q // QT,)
    return pl.pallas_call(
        _decode_body,
        grid=grid,
        in_specs=[
            pl.BlockSpec((QT, W), lambda i: (i, 0)),
            pl.BlockSpec((H, S), lambda i: (0, 0)),
            pl.BlockSpec((1, S), lambda i: (0, 0)),
            pl.BlockSpec((QT, 1), lambda i: (i, 0)),
        ],
        out_specs=[
            pl.BlockSpec((QT, 1), lambda i: (i, 0)),
            pl.BlockSpec((QT, NUM_LABELS), lambda i: (i, 0)),
        ],
        out_shape=[
            jax.ShapeDtypeStruct((Q, 1), jnp.int32),
            jax.ShapeDtypeStruct((Q, NUM_LABELS), jnp.float32),
        ],
        scratch_shapes=[
            pltpu.VMEM((H, S), jnp.float32),
            pltpu.VMEM((1, S), jnp.float32),
        ],
        interpret=interpret,
    )(emb, sT, labels_f, qtok_f)


def kernel(support, label_support, query, E):
    support = support.astype(jnp.int32)
    qflat = query.astype(jnp.int32).reshape(-1)
    S = support.shape[0]
    H = E.shape[1]
    Q = qflat.shape[0]

    emb = _sc_gather_rows(E, qflat, support)

    sT = emb[Q:, :H].T
    labels_f = label_support.astype(jnp.float32).reshape(1, S)
    qtok_f = qflat.astype(jnp.float32).reshape(-1, 1)

    best, near = _decode(emb, sT, labels_f, qtok_f)
    return (best.reshape(query.shape),
            near.reshape(query.shape + (NUM_LABELS,)))


# speedup vs baseline: 1.2116x; 1.0528x over previous
import functools

import jax
import jax.numpy as jnp
from jax import lax
from jax.experimental import pallas as pl
from jax.experimental.pallas import tpu as pltpu
from jax.experimental.pallas import tpu_sc as plsc

NUM_LABELS = 32
PAD = 0
NEG = -1000000000.0
QT = 512


def _sc_gather(Epad, qtok, stok):
    info = plsc.get_sparse_core_info()
    NC, NS = info.num_cores, info.num_subcores
    NW = NC * NS
    Q = qtok.shape[0]
    S = stok.shape[0]
    D = Epad.shape[1]
    q_w = Q // NW
    s_w = S // NW
    QCH = q_w // 128

    mesh = plsc.VectorSubcoreMesh(core_axis_name="c", subcore_axis_name="s")
    chunk_sizes = [128] * QCH + [s_w]
    scratch = ([pltpu.VMEM((c,), jnp.int32) for c in chunk_sizes]
               + [pltpu.VMEM((c, D), jnp.float32) for c in chunk_sizes]
               + [pltpu.SemaphoreType.DMA])
    NCHUNK = len(chunk_sizes)

    @functools.partial(
        pl.kernel,
        mesh=mesh,
        out_type=jax.ShapeDtypeStruct((Q + S, D), jnp.float32),
        scratch_types=scratch,
    )
    def gather_kernel(table_hbm, qtok_hbm, stok_hbm, out_hbm, *refs):
        idx_refs = refs[:NCHUNK]
        row_refs = refs[NCHUNK:2 * NCHUNK]
        sem = refs[2 * NCHUNK]
        wid = lax.axis_index("s") * NC + lax.axis_index("c")
        qbase = wid * q_w
        sbase = wid * s_w
        srcs = ([(qtok_hbm, qbase + c * 128, qbase + c * 128)
                 for c in range(QCH)]
                + [(stok_hbm, sbase, Q + sbase)])
        for (src, off, _), cs, iref in zip(srcs, chunk_sizes, idx_refs):
            pltpu.sync_copy(src.at[pl.ds(off, cs)], iref)
        handles = [
            pltpu.async_copy(table_hbm.at[iref], rref, sem)
            for iref, rref in zip(idx_refs, row_refs)
        ]
        for h in handles:
            h.wait()
        for (_, _, out_off), cs, rref in zip(srcs, chunk_sizes, row_refs):
            pltpu.sync_copy(rref, out_hbm.at[pl.ds(out_off, cs)])

    return gather_kernel(Epad, qtok, stok)


def _sc_gather_rows(E, qtok, stok):
    info = plsc.get_sparse_core_info()
    NC, NS = info.num_cores, info.num_subcores
    NW = NC * NS
    Q = qtok.shape[0]
    S = stok.shape[0]
    H = E.shape[1]
    q_w = Q // NW
    s_w = S // NW
    n_w = q_w + s_w
    BATCH = 16

    mesh = plsc.VectorSubcoreMesh(core_axis_name="c", subcore_axis_name="s")
    scratch = [
        pltpu.VMEM((n_w,), jnp.int32),
        pltpu.VMEM((n_w, H), jnp.float32),
        pltpu.SemaphoreType.DMA,
        pltpu.SemaphoreType.DMA,
    ]

    @functools.partial(
        pl.kernel,
        mesh=mesh,
        out_type=jax.ShapeDtypeStruct((Q + S, H), jnp.float32),
        scratch_types=scratch,
    )
    def gather_kernel(table_hbm, qtok_hbm, stok_hbm, out_hbm, idx_v, rows_v,
                      sem0, sem1):
        wid = lax.axis_index("s") * NC + lax.axis_index("c")
        qbase = wid * q_w
        sbase = wid * s_w
        pltpu.sync_copy(qtok_hbm.at[pl.ds(qbase, q_w)],
                        idx_v.at[pl.ds(0, q_w)])
        pltpu.sync_copy(stok_hbm.at[pl.ds(sbase, s_w)],
                        idx_v.at[pl.ds(q_w, s_w)])
        sems = (sem0, sem1)
        pending = []
        for b in range(n_w // BATCH):
            sem = sems[b % 2]
            toks = idx_v[pl.ds(b * BATCH, BATCH)]
            batch = []
            for j in range(BATCH):
                i = b * BATCH + j
                batch.append(pltpu.async_copy(
                    table_hbm.at[pl.ds(toks[j], 1)],
                    rows_v.at[pl.ds(i, 1)], sem))
            for h in pending:
                h.wait()
            pending = batch
        for h in pending:
            h.wait()
        pltpu.sync_copy(rows_v.at[pl.ds(0, q_w)],
                        out_hbm.at[pl.ds(qbase, q_w)])
        pltpu.sync_copy(rows_v.at[pl.ds(q_w, s_w)],
                        out_hbm.at[pl.ds(Q + sbase, s_w)])

    return gather_kernel(E, qtok, stok)


def _decode_body(x_ref, sT_ref, lab_ref, qtok_ref, best_ref, near_ref,
                 ynT_s, y2_s):
    @pl.when(pl.program_id(0) == 0)
    def _():
        sT = sT_ref[...]
        ns = jnp.sqrt(jnp.sum(sT * sT, axis=0, keepdims=True))
        ynT = sT / jnp.maximum(ns, 1e-12)
        ynT_s[...] = ynT
        y2_s[...] = jnp.sum(ynT * ynT, axis=0, keepdims=True)

    H = sT_ref.shape[0]
    x = x_ref[:, :H]
    nx = jnp.sqrt(jnp.sum(x * x, axis=1, keepdims=True))
    xn = x / jnp.maximum(nx, 1e-12)
    x2 = jnp.sum(xn * xn, axis=1, keepdims=True)

    d = lax.dot_general(xn, ynT_s[...], (((1,), (0,)), ((), ())),
                        preferred_element_type=jnp.float32)
    scores = 2.0 * d - x2 - y2_s[...]

    lab = lab_ref[...]
    qv = qtok_ref[...] != float(PAD)
    lv = lab != float(PAD)
    scores = jnp.where(jnp.logical_and(qv, lv), scores, NEG)

    m = jnp.max(scores, axis=1, keepdims=True)
    iota = lax.broadcasted_iota(jnp.int32, scores.shape, 1)
    best = jnp.min(jnp.where(scores == m, iota, jnp.int32(2**30)),
                   axis=1, keepdims=True)
    bl = jnp.max(jnp.where(iota == best, lab, 0.0), axis=1, keepdims=True)
    best_ref[...] = bl.astype(jnp.int32)

    sbf = scores.astype(jnp.bfloat16)
    negb = jnp.bfloat16(NEG)
    cols = []
    for l in range(NUM_LABELS):
        sel = jnp.where(lab == float(l), sbf, negb)
        cols.append(jnp.max(sel, axis=1, keepdims=True))
    near = jnp.concatenate(cols, axis=1).astype(jnp.float32)
    near_ref[...] = jnp.where(near < NEG * 0.5, NEG, near)


def _decode(emb, sT, labels_f, qtok_f, interpret=False):
    Q = qtok_f.shape[0]
    W = emb.shape[1]
    H = sT.shape[0]
    S = sT.shape[1]
    grid = (Q // QT,)
    return pl.pallas_call(
        _decode_body,
        grid=grid,
        in_specs=[
            pl.BlockSpec((QT, W), lambda i: (i, 0)),
            pl.BlockSpec((H, S), lambda i: (0, 0)),
            pl.BlockSpec((1, S), lambda i: (0, 0)),
            pl.BlockSpec((QT, 1), lambda i: (i, 0)),
        ],
        out_specs=[
            pl.BlockSpec((QT, 1), lambda i: (i, 0)),
            pl.BlockSpec((QT, NUM_LABELS), lambda i: (i, 0)),
        ],
        out_shape=[
            jax.ShapeDtypeStruct((Q, 1), jnp.int32),
            jax.ShapeDtypeStruct((Q, NUM_LABELS), jnp.float32),
        ],
        scratch_shapes=[
            pltpu.VMEM((H, S), jnp.float32),
            pltpu.VMEM((1, S), jnp.float32),
        ],
        interpret=interpret,
    )(emb, sT, labels_f, qtok_f)


def kernel(support, label_support, query, E):
    support = support.astype(jnp.int32)
    qflat = query.astype(jnp.int32).reshape(-1)
    S = support.shape[0]
    H = E.shape[1]
    Q = qflat.shape[0]

    emb = _sc_gather_rows(E, qflat, support)

    sT = emb[Q:, :H].T
    labels_f = label_support.astype(jnp.float32).reshape(1, S)
    qtok_f = qflat.astype(jnp.float32).reshape(-1, 1)

    best, near = _decode(emb, sT, labels_f, qtok_f)
    return (best.reshape(query.shape),
            near.reshape(query.shape + (NUM_LABELS,)))


# speedup vs baseline: 1.2420x; 1.0251x over previous
import functools

import jax
import jax.numpy as jnp
from jax import lax
from jax.experimental import pallas as pl
from jax.experimental.pallas import tpu as pltpu
from jax.experimental.pallas import tpu_sc as plsc

NUM_LABELS = 32
PAD = 0
NEG = -1000000000.0
QT = 1024


def _sc_gather(Epad, qtok, stok):
    info = plsc.get_sparse_core_info()
    NC, NS = info.num_cores, info.num_subcores
    NW = NC * NS
    Q = qtok.shape[0]
    S = stok.shape[0]
    D = Epad.shape[1]
    q_w = Q // NW
    s_w = S // NW
    QCH = q_w // 128

    mesh = plsc.VectorSubcoreMesh(core_axis_name="c", subcore_axis_name="s")
    chunk_sizes = [128] * QCH + [s_w]
    scratch = ([pltpu.VMEM((c,), jnp.int32) for c in chunk_sizes]
               + [pltpu.VMEM((c, D), jnp.float32) for c in chunk_sizes]
               + [pltpu.SemaphoreType.DMA])
    NCHUNK = len(chunk_sizes)

    @functools.partial(
        pl.kernel,
        mesh=mesh,
        out_type=jax.ShapeDtypeStruct((Q + S, D), jnp.float32),
        scratch_types=scratch,
    )
    def gather_kernel(table_hbm, qtok_hbm, stok_hbm, out_hbm, *refs):
        idx_refs = refs[:NCHUNK]
        row_refs = refs[NCHUNK:2 * NCHUNK]
        sem = refs[2 * NCHUNK]
        wid = lax.axis_index("s") * NC + lax.axis_index("c")
        qbase = wid * q_w
        sbase = wid * s_w
        srcs = ([(qtok_hbm, qbase + c * 128, qbase + c * 128)
                 for c in range(QCH)]
                + [(stok_hbm, sbase, Q + sbase)])
        for (src, off, _), cs, iref in zip(srcs, chunk_sizes, idx_refs):
            pltpu.sync_copy(src.at[pl.ds(off, cs)], iref)
        handles = [
            pltpu.async_copy(table_hbm.at[iref], rref, sem)
            for iref, rref in zip(idx_refs, row_refs)
        ]
        for h in handles:
            h.wait()
        for (_, _, out_off), cs, rref in zip(srcs, chunk_sizes, row_refs):
            pltpu.sync_copy(rref, out_hbm.at[pl.ds(out_off, cs)])

    return gather_kernel(Epad, qtok, stok)


def _sc_gather_rows(E, qtok, stok):
    info = plsc.get_sparse_core_info()
    NC, NS = info.num_cores, info.num_subcores
    NW = NC * NS
    Q = qtok.shape[0]
    S = stok.shape[0]
    H = E.shape[1]
    q_w = Q // NW
    s_w = S // NW
    n_w = q_w + s_w
    BATCH = 16

    mesh = plsc.VectorSubcoreMesh(core_axis_name="c", subcore_axis_name="s")
    scratch = [
        pltpu.VMEM((n_w,), jnp.int32),
        pltpu.VMEM((n_w, H), jnp.float32),
        pltpu.SemaphoreType.DMA,
        pltpu.SemaphoreType.DMA,
    ]

    @functools.partial(
        pl.kernel,
        mesh=mesh,
        out_type=jax.ShapeDtypeStruct((Q + S, H), jnp.float32),
        scratch_types=scratch,
    )
    def gather_kernel(table_hbm, qtok_hbm, stok_hbm, out_hbm, idx_v, rows_v,
                      sem0, sem1):
        wid = lax.axis_index("s") * NC + lax.axis_index("c")
        qbase = wid * q_w
        sbase = wid * s_w
        pltpu.sync_copy(qtok_hbm.at[pl.ds(qbase, q_w)],
                        idx_v.at[pl.ds(0, q_w)])
        pltpu.sync_copy(stok_hbm.at[pl.ds(sbase, s_w)],
                        idx_v.at[pl.ds(q_w, s_w)])
        sems = (sem0, sem1)
        pending = []
        for b in range(n_w // BATCH):
            sem = sems[b % 2]
            toks = idx_v[pl.ds(b * BATCH, BATCH)]
            batch = []
            for j in range(BATCH):
                i = b * BATCH + j
                batch.append(pltpu.async_copy(
                    table_hbm.at[pl.ds(toks[j], 1)],
                    rows_v.at[pl.ds(i, 1)], sem))
            for h in pending:
                h.wait()
            pending = batch
        for h in pending:
            h.wait()
        pltpu.sync_copy(rows_v.at[pl.ds(0, q_w)],
                        out_hbm.at[pl.ds(qbase, q_w)])
        pltpu.sync_copy(rows_v.at[pl.ds(q_w, s_w)],
                        out_hbm.at[pl.ds(Q + sbase, s_w)])

    return gather_kernel(E, qtok, stok)


def _decode_body(x_ref, sT_ref, lab_ref, qtok_ref, best_ref, near_ref,
                 ynT_s, y2_s):
    @pl.when(pl.program_id(0) == 0)
    def _():
        sT = sT_ref[...]
        ns = jnp.sqrt(jnp.sum(sT * sT, axis=0, keepdims=True))
        ynT = sT / jnp.maximum(ns, 1e-12)
        ynT_s[...] = ynT
        y2_s[...] = jnp.sum(ynT * ynT, axis=0, keepdims=True)

    H = sT_ref.shape[0]
    x = x_ref[:, :H]
    nx = jnp.sqrt(jnp.sum(x * x, axis=1, keepdims=True))
    xn = x / jnp.maximum(nx, 1e-12)
    x2 = jnp.sum(xn * xn, axis=1, keepdims=True)

    d = lax.dot_general(xn, ynT_s[...], (((1,), (0,)), ((), ())),
                        preferred_element_type=jnp.float32)
    scores = 2.0 * d - x2 - y2_s[...]

    lab = lab_ref[...]
    qv = qtok_ref[...] != float(PAD)
    lv = lab != float(PAD)
    scores = jnp.where(jnp.logical_and(qv, lv), scores, NEG)

    m = jnp.max(scores, axis=1, keepdims=True)
    iota = lax.broadcasted_iota(jnp.int32, scores.shape, 1)
    best = jnp.min(jnp.where(scores == m, iota, jnp.int32(2**30)),
                   axis=1, keepdims=True)
    bl = jnp.max(jnp.where(iota == best, lab, 0.0), axis=1, keepdims=True)
    best_ref[...] = bl.astype(jnp.int32)

    sbf = scores.astype(jnp.bfloat16)
    negb = jnp.bfloat16(NEG)
    cols = []
    for l in range(NUM_LABELS):
        sel = jnp.where(lab == float(l), sbf, negb)
        cols.append(jnp.max(sel, axis=1, keepdims=True))
    near = jnp.concatenate(cols, axis=1).astype(jnp.float32)
    near_ref[...] = jnp.where(near < NEG * 0.5, NEG, near)


def _decode(emb, sT, labels_f, qtok_f, interpret=False):
    Q = qtok_f.shape[0]
    W = emb.shape[1]
    H = sT.shape[0]
    S = sT.shape[1]
    grid = (Q // QT,)
    return pl.pallas_call(
        _decode_body,
        grid=grid,
        in_specs=[
            pl.BlockSpec((QT, W), lambda i: (i, 0)),
            pl.BlockSpec((H, S), lambda i: (0, 0)),
            pl.BlockSpec((1, S), lambda i: (0, 0)),
            pl.BlockSpec((QT, 1), lambda i: (i, 0)),
        ],
        out_specs=[
            pl.BlockSpec((QT, 1), lambda i: (i, 0)),
            pl.BlockSpec((QT, NUM_LABELS), lambda i: (i, 0)),
        ],
        out_shape=[
            jax.ShapeDtypeStruct((Q, 1), jnp.int32),
            jax.ShapeDtypeStruct((Q, NUM_LABELS), jnp.float32),
        ],
        scratch_shapes=[
            pltpu.VMEM((H, S), jnp.float32),
            pltpu.VMEM((1, S), jnp.float32),
        ],
        interpret=interpret,
    )(emb, sT, labels_f, qtok_f)


def kernel(support, label_support, query, E):
    support = support.astype(jnp.int32)
    qflat = query.astype(jnp.int32).reshape(-1)
    S = support.shape[0]
    H = E.shape[1]
    Q = qflat.shape[0]

    emb = _sc_gather_rows(E, qflat, support)

    sT = emb[Q:, :H].T
    labels_f = label_support.astype(jnp.float32).reshape(1, S)
    qtok_f = qflat.astype(jnp.float32).reshape(-1, 1)

    best, near = _decode(emb, sT, labels_f, qtok_f)
    return (best.reshape(query.shape),
            near.reshape(query.shape + (NUM_LABELS,)))


# speedup vs baseline: 1.2562x; 1.0115x over previous
import functools

import jax
import jax.numpy as jnp
from jax import lax
from jax.experimental import pallas as pl
from jax.experimental.pallas import tpu as pltpu
from jax.experimental.pallas import tpu_sc as plsc

NUM_LABELS = 32
PAD = 0
NEG = -1000000000.0
QT = 2048


def _sc_gather(Epad, qtok, stok):
    info = plsc.get_sparse_core_info()
    NC, NS = info.num_cores, info.num_subcores
    NW = NC * NS
    Q = qtok.shape[0]
    S = stok.shape[0]
    D = Epad.shape[1]
    q_w = Q // NW
    s_w = S // NW
    QCH = q_w // 128

    mesh = plsc.VectorSubcoreMesh(core_axis_name="c", subcore_axis_name="s")
    chunk_sizes = [128] * QCH + [s_w]
    scratch = ([pltpu.VMEM((c,), jnp.int32) for c in chunk_sizes]
               + [pltpu.VMEM((c, D), jnp.float32) for c in chunk_sizes]
               + [pltpu.SemaphoreType.DMA])
    NCHUNK = len(chunk_sizes)

    @functools.partial(
        pl.kernel,
        mesh=mesh,
        out_type=jax.ShapeDtypeStruct((Q + S, D), jnp.float32),
        scratch_types=scratch,
    )
    def gather_kernel(table_hbm, qtok_hbm, stok_hbm, out_hbm, *refs):
        idx_refs = refs[:NCHUNK]
        row_refs = refs[NCHUNK:2 * NCHUNK]
        sem = refs[2 * NCHUNK]
        wid = lax.axis_index("s") * NC + lax.axis_index("c")
        qbase = wid * q_w
        sbase = wid * s_w
        srcs = ([(qtok_hbm, qbase + c * 128, qbase + c * 128)
                 for c in range(QCH)]
                + [(stok_hbm, sbase, Q + sbase)])
        for (src, off, _), cs, iref in zip(srcs, chunk_sizes, idx_refs):
            pltpu.sync_copy(src.at[pl.ds(off, cs)], iref)
        handles = [
            pltpu.async_copy(table_hbm.at[iref], rref, sem)
            for iref, rref in zip(idx_refs, row_refs)
        ]
        for h in handles:
            h.wait()
        for (_, _, out_off), cs, rref in zip(srcs, chunk_sizes, row_refs):
            pltpu.sync_copy(rref, out_hbm.at[pl.ds(out_off, cs)])

    return gather_kernel(Epad, qtok, stok)


def _sc_gather_rows(E, qtok, stok):
    info = plsc.get_sparse_core_info()
    NC, NS = info.num_cores, info.num_subcores
    NW = NC * NS
    Q = qtok.shape[0]
    S = stok.shape[0]
    H = E.shape[1]
    q_w = Q // NW
    s_w = S // NW
    n_w = q_w + s_w
    BATCH = 16

    mesh = plsc.VectorSubcoreMesh(core_axis_name="c", subcore_axis_name="s")
    scratch = [
        pltpu.VMEM((n_w,), jnp.int32),
        pltpu.VMEM((n_w, H), jnp.float32),
        pltpu.SemaphoreType.DMA,
        pltpu.SemaphoreType.DMA,
    ]

    @functools.partial(
        pl.kernel,
        mesh=mesh,
        out_type=jax.ShapeDtypeStruct((Q + S, H), jnp.float32),
        scratch_types=scratch,
    )
    def gather_kernel(table_hbm, qtok_hbm, stok_hbm, out_hbm, idx_v, rows_v,
                      sem0, sem1):
        wid = lax.axis_index("s") * NC + lax.axis_index("c")
        qbase = wid * q_w
        sbase = wid * s_w
        pltpu.sync_copy(qtok_hbm.at[pl.ds(qbase, q_w)],
                        idx_v.at[pl.ds(0, q_w)])
        pltpu.sync_copy(stok_hbm.at[pl.ds(sbase, s_w)],
                        idx_v.at[pl.ds(q_w, s_w)])
        sems = (sem0, sem1)
        pending = []
        for b in range(n_w // BATCH):
            sem = sems[b % 2]
            toks = idx_v[pl.ds(b * BATCH, BATCH)]
            batch = []
            for j in range(BATCH):
                i = b * BATCH + j
                batch.append(pltpu.async_copy(
                    table_hbm.at[pl.ds(toks[j], 1)],
                    rows_v.at[pl.ds(i, 1)], sem))
            for h in pending:
                h.wait()
            pending = batch
        for h in pending:
            h.wait()
        pltpu.sync_copy(rows_v.at[pl.ds(0, q_w)],
                        out_hbm.at[pl.ds(qbase, q_w)])
        pltpu.sync_copy(rows_v.at[pl.ds(q_w, s_w)],
                        out_hbm.at[pl.ds(Q + sbase, s_w)])

    return gather_kernel(E, qtok, stok)


def _decode_body(x_ref, sT_ref, lab_ref, qtok_ref, best_ref, near_ref,
                 ynT_s, y2_s):
    @pl.when(pl.program_id(0) == 0)
    def _():
        sT = sT_ref[...]
        ns = jnp.sqrt(jnp.sum(sT * sT, axis=0, keepdims=True))
        ynT = sT / jnp.maximum(ns, 1e-12)
        ynT_s[...] = ynT
        y2_s[...] = jnp.sum(ynT * ynT, axis=0, keepdims=True)

    H = sT_ref.shape[0]
    x = x_ref[:, :H]
    nx = jnp.sqrt(jnp.sum(x * x, axis=1, keepdims=True))
    xn = x / jnp.maximum(nx, 1e-12)
    x2 = jnp.sum(xn * xn, axis=1, keepdims=True)

    d = lax.dot_general(xn, ynT_s[...], (((1,), (0,)), ((), ())),
                        preferred_element_type=jnp.float32)
    scores = 2.0 * d - x2 - y2_s[...]

    lab = lab_ref[...]
    qv = qtok_ref[...] != float(PAD)
    lv = lab != float(PAD)
    scores = jnp.where(jnp.logical_and(qv, lv), scores, NEG)

    m = jnp.max(scores, axis=1, keepdims=True)
    iota = lax.broadcasted_iota(jnp.int32, scores.shape, 1)
    best = jnp.min(jnp.where(scores == m, iota, jnp.int32(2**30)),
                   axis=1, keepdims=True)
    bl = jnp.max(jnp.where(iota == best, lab, 0.0), axis=1, keepdims=True)
    best_ref[...] = bl.astype(jnp.int32)

    sbf = scores.astype(jnp.bfloat16)
    negb = jnp.bfloat16(NEG)
    cols = []
    for l in range(NUM_LABELS):
        sel = jnp.where(lab == float(l), sbf, negb)
        cols.append(jnp.max(sel, axis=1, keepdims=True))
    near = jnp.concatenate(cols, axis=1).astype(jnp.float32)
    near_ref[...] = jnp.where(near < NEG * 0.5, NEG, near)


def _decode(emb, sT, labels_f, qtok_f, interpret=False):
    Q = qtok_f.shape[0]
    W = emb.shape[1]
    H = sT.shape[0]
    S = sT.shape[1]
    grid = (Q // QT,)
    return pl.pallas_call(
        _decode_body,
        grid=grid,
        in_specs=[
            pl.BlockSpec((QT, W), lambda i: (i, 0)),
            pl.BlockSpec((H, S), lambda i: (0, 0)),
            pl.BlockSpec((1, S), lambda i: (0, 0)),
            pl.BlockSpec((QT, 1), lambda i: (i, 0)),
        ],
        out_specs=[
            pl.BlockSpec((QT, 1), lambda i: (i, 0)),
            pl.BlockSpec((QT, NUM_LABELS), lambda i: (i, 0)),
        ],
        out_shape=[
            jax.ShapeDtypeStruct((Q, 1), jnp.int32),
            jax.ShapeDtypeStruct((Q, NUM_LABELS), jnp.float32),
        ],
        scratch_shapes=[
            pltpu.VMEM((H, S), jnp.float32),
            pltpu.VMEM((1, S), jnp.float32),
        ],
        interpret=interpret,
    )(emb, sT, labels_f, qtok_f)


def kernel(support, label_support, query, E):
    support = support.astype(jnp.int32)
    qflat = query.astype(jnp.int32).reshape(-1)
    S = support.shape[0]
    H = E.shape[1]
    Q = qflat.shape[0]

    emb = _sc_gather_rows(E, qflat, support)

    sT = emb[Q:, :H].T
    labels_f = label_support.astype(jnp.float32).reshape(1, S)
    qtok_f = qflat.astype(jnp.float32).reshape(-1, 1)

    best, near = _decode(emb, sT, labels_f, qtok_f)
    return (best.reshape(query.shape),
            near.reshape(query.shape + (NUM_LABELS,)))
